# baseline (speedup 1.0000x reference)
import jax
import jax.numpy as jnp
from jax.experimental import pallas as pl
from jax.experimental.pallas import tpu as pltpu

_CHUNK_ROWS = 8000
_NBUF = 12
_LAG = 6


def _copy_body(x_ref, o_ref, buf, in_sems, out_sems):
    rows = x_ref.shape[0]
    nch = rows // _CHUNK_ROWS

    def in_copy(i, b):
        return pltpu.make_async_copy(
            x_ref.at[pl.ds(i * _CHUNK_ROWS, _CHUNK_ROWS), :],
            buf.at[b],
            in_sems.at[b],
        )

    def out_copy(i, b):
        return pltpu.make_async_copy(
            buf.at[b],
            o_ref.at[pl.ds(i * _CHUNK_ROWS, _CHUNK_ROWS), :],
            out_sems.at[b],
        )

    for i in range(nch):
        b = i % _NBUF
        if i >= _NBUF:
            out_copy(i - _NBUF, b).wait()
        in_copy(i, b).start()
        j = i - _LAG
        if j >= 0:
            bj = j % _NBUF
            in_copy(j, bj).wait()
            out_copy(j, bj).start()
    for j in range(max(0, nch - _LAG), nch):
        bj = j % _NBUF
        in_copy(j, bj).wait()
        out_copy(j, bj).start()
    for j in range(max(0, nch - _NBUF), nch):
        out_copy(j, j % _NBUF).wait()


def kernel(table):
    rows, cols = table.shape
    return pl.pallas_call(
        _copy_body,
        out_shape=jax.ShapeDtypeStruct(table.shape, table.dtype),
        in_specs=[pl.BlockSpec(memory_space=pl.ANY)],
        out_specs=pl.BlockSpec(memory_space=pl.ANY),
        scratch_shapes=[
            pltpu.VMEM((_NBUF, _CHUNK_ROWS, cols), table.dtype),
            pltpu.SemaphoreType.DMA((_NBUF,)),
            pltpu.SemaphoreType.DMA((_NBUF,)),
        ],
    )(table)


# speedup vs baseline: 1.3123x; 1.3123x over previous
import jax
import jax.numpy as jnp
from jax.experimental import pallas as pl
from jax.experimental.pallas import tpu as pltpu

_OUTER = 125
_CHUNK = 64
_NBUF = 12
_LAG = 6


def _copy_body(x_ref, o_ref, buf, in_sems, out_sems):
    inner = x_ref.shape[1]
    nch = inner // _CHUNK

    def in_copy(i, b):
        return pltpu.make_async_copy(
            x_ref.at[:, pl.ds(i * _CHUNK, _CHUNK), :],
            buf.at[b],
            in_sems.at[b],
        )

    def out_copy(i, b):
        return pltpu.make_async_copy(
            buf.at[b],
            o_ref.at[:, pl.ds(i * _CHUNK, _CHUNK), :],
            out_sems.at[b],
        )

    for i in range(nch):
        b = i % _NBUF
        if i >= _NBUF:
            out_copy(i - _NBUF, b).wait()
        in_copy(i, b).start()
        j = i - _LAG
        if j >= 0:
            bj = j % _NBUF
            in_copy(j, bj).wait()
            out_copy(j, bj).start()
    for j in range(max(0, nch - _LAG), nch):
        bj = j % _NBUF
        in_copy(j, bj).wait()
        out_copy(j, bj).start()
    for j in range(max(0, nch - _NBUF), nch):
        out_copy(j, j % _NBUF).wait()


def kernel(table):
    rows, cols = table.shape
    inner = rows // _OUTER
    t = table.reshape(_OUTER, inner, cols)
    out = pl.pallas_call(
        _copy_body,
        out_shape=jax.ShapeDtypeStruct((_OUTER, inner, cols), table.dtype),
        in_specs=[pl.BlockSpec(memory_space=pl.ANY)],
        out_specs=pl.BlockSpec(memory_space=pl.ANY),
        scratch_shapes=[
            pltpu.VMEM((_NBUF, _OUTER, _CHUNK, cols), table.dtype),
            pltpu.SemaphoreType.DMA((_NBUF,)),
            pltpu.SemaphoreType.DMA((_NBUF,)),
        ],
    )(t)
    return out.reshape(rows, cols)


# speedup vs baseline: 1.3139x; 1.0012x over previous
import jax
import jax.numpy as jnp
from jax.experimental import pallas as pl
from jax.experimental.pallas import tpu as pltpu

_OUTER = 125
_CHUNK = 16
_NBUF = 32
_LAG = 16


def _copy_body(x_ref, o_ref, buf, in_sems, out_sems):
    inner = x_ref.shape[1]
    nch = inner // _CHUNK

    def in_copy(i, b):
        return pltpu.make_async_copy(
            x_ref.at[:, pl.ds(i * _CHUNK, _CHUNK), :],
            buf.at[b],
            in_sems.at[b],
        )

    def out_copy(i, b):
        return pltpu.make_async_copy(
            buf.at[b],
            o_ref.at[:, pl.ds(i * _CHUNK, _CHUNK), :],
            out_sems.at[b],
        )

    for i in range(nch):
        b = i % _NBUF
        if i >= _NBUF:
            out_copy(i - _NBUF, b).wait()
        in_copy(i, b).start()
        j = i - _LAG
        if j >= 0:
            bj = j % _NBUF
            in_copy(j, bj).wait()
            out_copy(j, bj).start()
    for j in range(max(0, nch - _LAG), nch):
        bj = j % _NBUF
        in_copy(j, bj).wait()
        out_copy(j, bj).start()
    for j in range(max(0, nch - _NBUF), nch):
        out_copy(j, j % _NBUF).wait()


def kernel(table):
    rows, cols = table.shape
    inner = rows // _OUTER
    t = table.reshape(_OUTER, inner, cols)
    out = pl.pallas_call(
        _copy_body,
        out_shape=jax.ShapeDtypeStruct((_OUTER, inner, cols), table.dtype),
        in_specs=[pl.BlockSpec(memory_space=pl.ANY)],
        out_specs=pl.BlockSpec(memory_space=pl.ANY),
        scratch_shapes=[
            pltpu.VMEM((_NBUF, _OUTER, _CHUNK, cols), table.dtype),
            pltpu.SemaphoreType.DMA((_NBUF,)),
            pltpu.SemaphoreType.DMA((_NBUF,)),
        ],
    )(t)
    return out.reshape(rows, cols)


# speedup vs baseline: 1.3152x; 1.0010x over previous
import jax
import jax.numpy as jnp
from jax.experimental import pallas as pl
from jax.experimental.pallas import tpu as pltpu

_OUTER = 125
_CHUNK = 16
_NBUF = 32
_LAG = 16


def _copy_body(x_ref, o_ref, buf, in_sems, out_sems):
    inner = x_ref.shape[1]
    nch = inner // _CHUNK

    def in_copy(i, b):
        return pltpu.make_async_copy(
            x_ref.at[:, pl.ds(i * _CHUNK, _CHUNK), :],
            buf.at[b],
            in_sems.at[b],
        )

    def out_copy(i, b):
        return pltpu.make_async_copy(
            buf.at[b],
            o_ref.at[:, pl.ds(i * _CHUNK, _CHUNK), :],
            out_sems.at[b],
        )

    for i in range(nch):
        b = i % _NBUF
        if i >= _NBUF:
            out_copy(i - _NBUF, b).wait()
        in_copy(i, b).start(priority=i % 2)
        j = i - _LAG
        if j >= 0:
            bj = j % _NBUF
            in_copy(j, bj).wait()
            out_copy(j, bj).start(priority=j % 2)
    for j in range(max(0, nch - _LAG), nch):
        bj = j % _NBUF
        in_copy(j, bj).wait()
        out_copy(j, bj).start(priority=j % 2)
    for j in range(max(0, nch - _NBUF), nch):
        out_copy(j, j % _NBUF).wait()


def kernel(table):
    rows, cols = table.shape
    inner = rows // _OUTER
    t = table.reshape(_OUTER, inner, cols)
    out = pl.pallas_call(
        _copy_body,
        out_shape=jax.ShapeDtypeStruct((_OUTER, inner, cols), table.dtype),
        in_specs=[pl.BlockSpec(memory_space=pl.ANY)],
        out_specs=pl.BlockSpec(memory_space=pl.ANY),
        scratch_shapes=[
            pltpu.VMEM((_NBUF, _OUTER, _CHUNK, cols), table.dtype),
            pltpu.SemaphoreType.DMA((_NBUF,)),
            pltpu.SemaphoreType.DMA((_NBUF,)),
        ],
    )(t)
    return out.reshape(rows, cols)
